# baseline (speedup 1.0000x reference)
import jax
import jax.numpy as jnp
from jax.experimental import pallas as pl
from jax.experimental.pallas import tpu as pltpu

_BUFFER_SIZE = 4194304
_TAIL = 1536
_BUF_LEN = _BUFFER_SIZE + _TAIL
_NCHUNK = 8
_CHUNK = _BUFFER_SIZE // _NCHUNK


def _copy_kernel(x_ref, buf_ref, out_ref, sems, tail_sem):
    tail = pltpu.make_async_copy(
        buf_ref.at[pl.ds(_BUFFER_SIZE, _TAIL)],
        out_ref.at[pl.ds(0, _TAIL)],
        tail_sem,
    )
    tail.start()
    copies = []
    for i in range(_NCHUNK):
        c = pltpu.make_async_copy(
            x_ref.at[pl.ds(i * _CHUNK, _CHUNK)],
            out_ref.at[pl.ds(_TAIL + i * _CHUNK, _CHUNK)],
            sems.at[i],
        )
        c.start()
        copies.append(c)
    tail.wait()
    for c in copies:
        c.wait()


def kernel(x, buffer):
    out = pl.pallas_call(
        _copy_kernel,
        out_shape=jax.ShapeDtypeStruct((_BUF_LEN,), jnp.float32),
        in_specs=[
            pl.BlockSpec(memory_space=pl.ANY),
            pl.BlockSpec(memory_space=pl.ANY),
        ],
        out_specs=pl.BlockSpec(memory_space=pl.ANY),
        scratch_shapes=[
            pltpu.SemaphoreType.DMA((_NCHUNK,)),
            pltpu.SemaphoreType.DMA,
        ],
    )(x.reshape(_BUFFER_SIZE), buffer.reshape(_BUF_LEN))
    return out.reshape(1, _BUF_LEN)


# speedup vs baseline: 2.9510x; 2.9510x over previous
import jax
import jax.numpy as jnp
from jax.experimental import pallas as pl
from jax.experimental.pallas import tpu as pltpu

_BUFFER_SIZE = 4194304
_TAIL = 1536
_BUF_LEN = _BUFFER_SIZE + _TAIL
_B = 524288
_NBLK_X = _BUFFER_SIZE // _B
_GRID = _NBLK_X + 1


def _concat_kernel(tail_ref, x_ref, out_ref, carry_ref):
    i = pl.program_id(0)

    @pl.when(i == 0)
    def _():
        out_ref[pl.ds(0, _TAIL)] = tail_ref[:]

    @pl.when(i > 0)
    def _():
        out_ref[pl.ds(0, _TAIL)] = carry_ref[:]

    out_ref[pl.ds(_TAIL, _B - _TAIL)] = x_ref[pl.ds(0, _B - _TAIL)]
    carry_ref[:] = x_ref[pl.ds(_B - _TAIL, _TAIL)]


def kernel(x, buffer):
    xf = x.reshape(_BUFFER_SIZE)
    tail = buffer.reshape(_BUF_LEN)[_BUFFER_SIZE:]
    out = pl.pallas_call(
        _concat_kernel,
        grid=(_GRID,),
        out_shape=jax.ShapeDtypeStruct((_BUF_LEN,), jnp.float32),
        in_specs=[
            pl.BlockSpec((_TAIL,), lambda i: 0),
            pl.BlockSpec((_B,), lambda i: jnp.minimum(i, _NBLK_X - 1)),
        ],
        out_specs=pl.BlockSpec((_B,), lambda i: i),
        scratch_shapes=[pltpu.VMEM((_TAIL,), jnp.float32)],
    )(tail, xf)
    return out.reshape(1, _BUF_LEN)
